# baseline (speedup 1.0000x reference)
import functools

import jax
import jax.numpy as jnp
from jax import lax
from jax.experimental import pallas as pl
from jax.experimental.pallas import tpu as pltpu
from jax.experimental.pallas import tpu_sc as plsc

_N = 10000
_E = 320000
_D = 128

_NC = 2
_NS = 16
_NW = _NC * _NS
_EPW = _E // _NW
_CH = 80
_NCH = _EPW // _CH
_NPAD = 10240
_RPT = _NPAD // _NS

_RBLK = 1000
_NBLK = _N // _RBLK


def _prologue_body(h_ref, wl_ref, bl_ref, w1a_ref, w1b_ref, b1_ref,
                   x_ref, a_ref, bx_ref):
  x = jnp.dot(h_ref[...], wl_ref[...],
              preferred_element_type=jnp.float32) + bl_ref[...]
  x_ref[...] = x
  a_ref[...] = jnp.dot(x, w1a_ref[...],
                       preferred_element_type=jnp.float32) + b1_ref[...]
  b = jnp.dot(x, w1b_ref[...], preferred_element_type=jnp.float32)
  bx_ref[...] = jnp.concatenate([b, x], axis=1)


def _prologue(h, wl, bl, w1a, w1b, b1):
  return pl.pallas_call(
      _prologue_body,
      grid=(_NBLK,),
      in_specs=[
          pl.BlockSpec((_RBLK, _D), lambda i: (i, 0)),
          pl.BlockSpec((_D, _D), lambda i: (0, 0)),
          pl.BlockSpec((1, _D), lambda i: (0, 0)),
          pl.BlockSpec((_D, _D), lambda i: (0, 0)),
          pl.BlockSpec((_D, _D), lambda i: (0, 0)),
          pl.BlockSpec((1, _D), lambda i: (0, 0)),
      ],
      out_specs=[
          pl.BlockSpec((_RBLK, _D), lambda i: (i, 0)),
          pl.BlockSpec((_RBLK, _D), lambda i: (i, 0)),
          pl.BlockSpec((_RBLK, 2 * _D), lambda i: (i, 0)),
      ],
      out_shape=[
          jax.ShapeDtypeStruct((_N, _D), jnp.float32),
          jax.ShapeDtypeStruct((_N, _D), jnp.float32),
          jax.ShapeDtypeStruct((_N, 2 * _D), jnp.float32),
      ],
  )(h, wl, bl, w1a, w1b, b1)


_PKW = _CH * 2 + _CH * 16


def _sc_body(pk_h, rowf_h, a_h, bx_h, par_h, zer_h, out_h,
             pk_v0, pk_v1, ar0, ar1, bxr0, bxr1, rsc, agg, par_v, acc_sh,
             sem_k0, sem_k1, sem_ga0, sem_ga1, sem_gb0, sem_gb1,
             sem_rs, sem_sc):
  pk_v = [pk_v0, pk_v1]
  arows = [ar0, ar1]
  bxrows = [bxr0, bxr1]
  sem_k = [sem_k0, sem_k1]
  sem_ga = [sem_ga0, sem_ga1]
  sem_gb = [sem_gb0, sem_gb1]

  c = lax.axis_index("c")
  s = lax.axis_index("s")
  wid = c * _NS + s

  pltpu.sync_copy(zer_h.at[pl.ds(s * _RPT, _RPT)],
                  acc_sh.at[pl.ds(s * _RPT, _RPT)])
  pltpu.sync_copy(par_h, par_v)
  plsc.subcore_barrier()

  wd = [par_v[pl.ds(16 * k, 16)] for k in range(8)]
  w2 = [par_v[pl.ds(128 + 16 * k, 16)] for k in range(8)]
  b2 = par_v[pl.ds(256, 16)][0]

  gbase = wid * _NCH
  ebase = wid * _EPW

  def issue_pk(ci, p):
    pltpu.async_copy(pk_h.at[pl.ds((gbase + ci) * _PKW, _PKW)], pk_v[p],
                     sem_k[p])

  def wait_pk(ci, p):
    pltpu.make_async_copy(pk_h.at[pl.ds((gbase + ci) * _PKW, _PKW)], pk_v[p],
                          sem_k[p]).wait()

  def issue_gather(p):
    pltpu.async_copy(a_h.at[pk_v[p].at[pl.ds(0, _CH)]], arows[p], sem_ga[p])
    pltpu.async_copy(bx_h.at[pk_v[p].at[pl.ds(_CH, _CH)]], bxrows[p],
                     sem_gb[p])

  def wait_gather(p):
    pltpu.make_async_copy(a_h.at[pk_v[p].at[pl.ds(0, _CH)]], arows[p],
                          sem_ga[p]).wait()
    pltpu.make_async_copy(bx_h.at[pk_v[p].at[pl.ds(_CH, _CH)]], bxrows[p],
                          sem_gb[p]).wait()

  def compute(p):
    av = arows[p]
    bv = bxrows[p]
    kv = pk_v[p]

    @plsc.parallel_loop(0, _CH, unroll=1)
    def _edge(e):
      dv = plsc.bitcast(kv[pl.ds(2 * _CH + 16 * e, 16)], jnp.float32)
      fr = []
      for k in range(8):
        pre = av[e, pl.ds(16 * k, 16)] + bv[e, pl.ds(16 * k, 16)] + dv * wd[k]
        pre = jnp.clip(pre, -10.0, 10.0)
        fr.append((pre * w2[k], 1.0 + jnp.exp(-pre)))
      while len(fr) > 1:
        fr = [(n1 * d2 + n2 * d1, d1 * d2)
              for (n1, d1), (n2, d2) in zip(fr[0::2], fr[1::2])]
      n, d = fr[0]
      t = jnp.sum(n / d) + b2
      attv = 1.0 / (1.0 + jnp.exp(jnp.full((16,), 0.0, jnp.float32) - t))
      for k2 in range(4):
        xa = bv[e, pl.ds(128 + 32 * k2, 16)] * attv
        xb = bv[e, pl.ds(128 + 32 * k2 + 16, 16)] * attv
        agg[e, pl.ds(32 * k2, 32)] = plsc.pack(
            xa, xb, format=plsc.PackFormat.INTERLEAVED)

  def wait_sc():
    pltpu.make_async_copy(agg, acc_sh.at[rsc], sem_sc).wait()

  issue_pk(0, 0)
  wait_pk(0, 0)
  issue_gather(0)
  issue_pk(1, 1)

  @pl.loop(0, _NCH // 2)
  def _outer(co):
    for b in range(2):
      p = b
      q = 1 - b
      ci = 2 * co + b

      @pl.when(ci >= 1)
      def _():
        wait_sc()

      pltpu.async_copy(rowf_h.at[pl.ds(ebase + ci * _CH, _CH)], rsc, sem_rs)

      @pl.when(ci + 1 < _NCH)
      def _():
        wait_pk(ci + 1, q)
        issue_gather(q)

      wait_gather(p)
      compute(p)
      pltpu.make_async_copy(rowf_h.at[pl.ds(ebase + ci * _CH, _CH)], rsc,
                            sem_rs).wait()
      pltpu.async_copy(agg, acc_sh.at[rsc], sem_sc, add=True)

      @pl.when(ci + 2 < _NCH)
      def _():
        issue_pk(ci + 2, p)

  wait_sc()
  plsc.subcore_barrier()
  pltpu.sync_copy(acc_sh.at[pl.ds(s * _RPT, _RPT)],
                  out_h.at[pl.ds(c * _NPAD + s * _RPT, _RPT)])


@functools.cache
def _make_sc_edge():
  return pl.kernel(
      _sc_body,
      out_type=jax.ShapeDtypeStruct((_NC * _NPAD, _D), jnp.bfloat16),
      mesh=plsc.VectorSubcoreMesh(core_axis_name="c", subcore_axis_name="s"),
      compiler_params=pltpu.CompilerParams(needs_layout_passes=False, use_tc_tiling_on_sc=False),
      scratch_types=[
          pltpu.VMEM((_PKW,), jnp.int32),
          pltpu.VMEM((_PKW,), jnp.int32),
          pltpu.VMEM((_CH, _D), jnp.float32),
          pltpu.VMEM((_CH, _D), jnp.float32),
          pltpu.VMEM((_CH, 2 * _D), jnp.float32),
          pltpu.VMEM((_CH, 2 * _D), jnp.float32),
          pltpu.VMEM((_CH,), jnp.int32),
          pltpu.VMEM((_CH, _D), jnp.bfloat16),
          pltpu.VMEM((272,), jnp.float32),
          pltpu.VMEM_SHARED((_NPAD, _D), jnp.bfloat16),
          pltpu.SemaphoreType.DMA,
          pltpu.SemaphoreType.DMA,
          pltpu.SemaphoreType.DMA,
          pltpu.SemaphoreType.DMA,
          pltpu.SemaphoreType.DMA,
          pltpu.SemaphoreType.DMA,
          pltpu.SemaphoreType.DMA,
          pltpu.SemaphoreType.DMA,
      ],
  )


def _epilogue_body(p0_ref, p1_ref, x_ref, pm_ref, g_ref, b_ref, o_ref):
  p = (p0_ref[...] + p1_ref[...]).astype(jnp.float32)
  o = jnp.dot(p, pm_ref[...], preferred_element_type=jnp.float32) * 0.01 \
      + x_ref[...]
  mean = jnp.mean(o, axis=1, keepdims=True)
  co = o - mean
  var = jnp.mean(co * co, axis=1, keepdims=True)
  ln = co * jax.lax.rsqrt(var + 1e-5) * g_ref[...] + b_ref[...]
  o_ref[...] = ln / (1.0 + jnp.exp(-ln))


def _epilogue(p0, p1, x, pm, gamma, beta):
  return pl.pallas_call(
      _epilogue_body,
      grid=(_NBLK,),
      in_specs=[
          pl.BlockSpec((_RBLK, _D), lambda i: (i, 0)),
          pl.BlockSpec((_RBLK, _D), lambda i: (i, 0)),
          pl.BlockSpec((_RBLK, _D), lambda i: (i, 0)),
          pl.BlockSpec((_D, _D), lambda i: (0, 0)),
          pl.BlockSpec((1, _D), lambda i: (0, 0)),
          pl.BlockSpec((1, _D), lambda i: (0, 0)),
      ],
      out_specs=pl.BlockSpec((_RBLK, _D), lambda i: (i, 0)),
      out_shape=jax.ShapeDtypeStruct((_N, _D), jnp.float32),
  )(p0, p1, x, pm, gamma, beta)


def kernel(h, distances, edges, node_mask, edge_mask, W_lin, b_lin, W1, b1,
           W2, b2, gamma, beta):
  x, a, bx = _prologue(h, W_lin, b_lin.reshape(1, _D), W1[:_D], W1[_D:2 * _D],
                       b1.reshape(1, _D))
  row = edges[0].astype(jnp.int32)
  col = edges[1].astype(jnp.int32)
  db = jnp.broadcast_to(distances, (_E, 16))
  dbi = jax.lax.bitcast_convert_type(db, jnp.int32)
  pk = jnp.concatenate(
      [row.reshape(-1, _CH), col.reshape(-1, _CH),
       dbi.reshape(-1, _CH * 16)], axis=1).reshape(-1)
  params = jnp.concatenate(
      [W1[2 * _D], W2[:, 0], jnp.pad(b2, (0, 15))]).astype(jnp.float32)
  zeros = jnp.zeros((_NPAD, _D), jnp.bfloat16)
  parts = _make_sc_edge()(pk, row, a, bx, params, zeros)
  blk = jnp.arange(128) // 32
  pos = jnp.arange(128) % 32
  feat = blk * 32 + jnp.where(pos % 2 == 0, pos // 2, 16 + pos // 2)
  pm = jax.nn.one_hot(feat, _D, dtype=jnp.float32)
  h_out = _epilogue(parts[:_N], parts[_NPAD:_NPAD + _N], x, pm,
                    gamma.reshape(1, _D), beta.reshape(1, _D))
  return (h_out, distances, edges, node_mask, edge_mask)


# speedup vs baseline: 1.0702x; 1.0702x over previous
import functools

import jax
import jax.numpy as jnp
from jax import lax
from jax.experimental import pallas as pl
from jax.experimental.pallas import tpu as pltpu
from jax.experimental.pallas import tpu_sc as plsc

_N = 10000
_E = 320000
_D = 128

_NC = 2
_NS = 16
_NW = _NC * _NS
_EPW = _E // _NW
_CH = 80
_NCH = _EPW // _CH
_NPAD = 10240
_RPT = _NPAD // _NS

_RBLK = 1000
_NBLK = _N // _RBLK


def _prologue_body(h_ref, wl_ref, bl_ref, w1a_ref, w1b_ref, b1_ref,
                   x_ref, a_ref, bx_ref):
  x = jnp.dot(h_ref[...], wl_ref[...],
              preferred_element_type=jnp.float32) + bl_ref[...]
  x_ref[...] = x
  a_ref[...] = jnp.dot(x, w1a_ref[...],
                       preferred_element_type=jnp.float32) + b1_ref[...]
  b = jnp.dot(x, w1b_ref[...], preferred_element_type=jnp.float32)
  bx_ref[...] = jnp.concatenate([b, x], axis=1)


def _prologue(h, wl, bl, w1a, w1b, b1):
  return pl.pallas_call(
      _prologue_body,
      grid=(_NBLK,),
      in_specs=[
          pl.BlockSpec((_RBLK, _D), lambda i: (i, 0)),
          pl.BlockSpec((_D, _D), lambda i: (0, 0)),
          pl.BlockSpec((1, _D), lambda i: (0, 0)),
          pl.BlockSpec((_D, _D), lambda i: (0, 0)),
          pl.BlockSpec((_D, _D), lambda i: (0, 0)),
          pl.BlockSpec((1, _D), lambda i: (0, 0)),
      ],
      out_specs=[
          pl.BlockSpec((_RBLK, _D), lambda i: (i, 0)),
          pl.BlockSpec((_RBLK, _D), lambda i: (i, 0)),
          pl.BlockSpec((_RBLK, 2 * _D), lambda i: (i, 0)),
      ],
      out_shape=[
          jax.ShapeDtypeStruct((_N, _D), jnp.float32),
          jax.ShapeDtypeStruct((_N, _D), jnp.float32),
          jax.ShapeDtypeStruct((_N, 2 * _D), jnp.float32),
      ],
  )(h, wl, bl, w1a, w1b, b1)


_PKW = _CH * 2 + _CH * 16


def _sc_body(pk_h, rowf_h, a_h, bx_h, par_h, zer_h, out_h,
             pk_v0, pk_v1, ar0, ar1, bxr0, bxr1, rsc, agg, par_v, acc_sh,
             sem_k0, sem_k1, sem_ga0, sem_ga1, sem_gb0, sem_gb1,
             sem_rs, sem_sc):
  pk_v = [pk_v0, pk_v1]
  arows = [ar0, ar1]
  bxrows = [bxr0, bxr1]
  sem_k = [sem_k0, sem_k1]
  sem_ga = [sem_ga0, sem_ga1]
  sem_gb = [sem_gb0, sem_gb1]

  c = lax.axis_index("c")
  s = lax.axis_index("s")
  wid = c * _NS + s

  pltpu.sync_copy(zer_h.at[pl.ds(s * _RPT, _RPT)],
                  acc_sh.at[pl.ds(s * _RPT, _RPT)])
  pltpu.sync_copy(par_h, par_v)
  plsc.subcore_barrier()

  wd = [par_v[pl.ds(16 * k, 16)] for k in range(8)]
  w2 = [par_v[pl.ds(128 + 16 * k, 16)] for k in range(8)]
  b2 = par_v[pl.ds(256, 16)][0]

  gbase = wid * _NCH
  ebase = wid * _EPW

  def issue_pk(ci, p):
    pltpu.async_copy(pk_h.at[pl.ds((gbase + ci) * _PKW, _PKW)], pk_v[p],
                     sem_k[p])

  def wait_pk(ci, p):
    pltpu.make_async_copy(pk_h.at[pl.ds((gbase + ci) * _PKW, _PKW)], pk_v[p],
                          sem_k[p]).wait()

  _H = _CH // 2

  def issue_gather(p):
    pltpu.async_copy(a_h.at[pk_v[p].at[pl.ds(0, _H)]],
                     arows[p].at[pl.ds(0, _H)], sem_ga[p])
    pltpu.async_copy(a_h.at[pk_v[p].at[pl.ds(_H, _H)]],
                     arows[p].at[pl.ds(_H, _H)], sem_ga[p])
    pltpu.async_copy(bx_h.at[pk_v[p].at[pl.ds(_CH, _H)]],
                     bxrows[p].at[pl.ds(0, _H)], sem_gb[p])
    pltpu.async_copy(bx_h.at[pk_v[p].at[pl.ds(_CH + _H, _H)]],
                     bxrows[p].at[pl.ds(_H, _H)], sem_gb[p])

  def wait_gather(p):
    pltpu.make_async_copy(a_h.at[pk_v[p].at[pl.ds(0, _H)]],
                          arows[p].at[pl.ds(0, _H)], sem_ga[p]).wait()
    pltpu.make_async_copy(a_h.at[pk_v[p].at[pl.ds(_H, _H)]],
                          arows[p].at[pl.ds(_H, _H)], sem_ga[p]).wait()
    pltpu.make_async_copy(bx_h.at[pk_v[p].at[pl.ds(_CH, _H)]],
                          bxrows[p].at[pl.ds(0, _H)], sem_gb[p]).wait()
    pltpu.make_async_copy(bx_h.at[pk_v[p].at[pl.ds(_CH + _H, _H)]],
                          bxrows[p].at[pl.ds(_H, _H)], sem_gb[p]).wait()

  def compute(p):
    av = arows[p]
    bv = bxrows[p]
    kv = pk_v[p]

    @plsc.parallel_loop(0, _CH, unroll=1)
    def _edge(e):
      dv = plsc.bitcast(kv[pl.ds(2 * _CH + 16 * e, 16)], jnp.float32)
      acc0 = jnp.zeros((16,), jnp.float32)
      acc1 = jnp.zeros((16,), jnp.float32)
      for k in range(8):
        pre = av[e, pl.ds(16 * k, 16)] + bv[e, pl.ds(16 * k, 16)] + dv * wd[k]
        sl = pre / (1.0 + jnp.exp(-pre))
        if k % 2 == 0:
          acc0 = acc0 + sl * w2[k]
        else:
          acc1 = acc1 + sl * w2[k]
      t = jnp.sum(acc0 + acc1) + b2
      attv = 1.0 / (1.0 + jnp.exp(jnp.full((16,), 0.0, jnp.float32) - t))
      for k2 in range(4):
        xa = bv[e, pl.ds(128 + 32 * k2, 16)] * attv
        xb = bv[e, pl.ds(128 + 32 * k2 + 16, 16)] * attv
        agg[e, pl.ds(32 * k2, 32)] = plsc.pack(
            xa, xb, format=plsc.PackFormat.INTERLEAVED)

  def wait_sc():
    pltpu.make_async_copy(agg, acc_sh.at[rsc], sem_sc).wait()

  issue_pk(0, 0)
  wait_pk(0, 0)
  issue_gather(0)
  issue_pk(1, 1)

  @pl.loop(0, _NCH // 2)
  def _outer(co):
    for b in range(2):
      p = b
      q = 1 - b
      ci = 2 * co + b

      @pl.when(ci >= 1)
      def _():
        wait_sc()

      pltpu.async_copy(rowf_h.at[pl.ds(ebase + ci * _CH, _CH)], rsc, sem_rs)

      @pl.when(ci + 1 < _NCH)
      def _():
        wait_pk(ci + 1, q)
        issue_gather(q)

      wait_gather(p)
      compute(p)
      pltpu.make_async_copy(rowf_h.at[pl.ds(ebase + ci * _CH, _CH)], rsc,
                            sem_rs).wait()
      pltpu.async_copy(agg, acc_sh.at[rsc], sem_sc, add=True)

      @pl.when(ci + 2 < _NCH)
      def _():
        issue_pk(ci + 2, p)

  wait_sc()
  plsc.subcore_barrier()
  pltpu.sync_copy(acc_sh.at[pl.ds(s * _RPT, _RPT)],
                  out_h.at[pl.ds(c * _NPAD + s * _RPT, _RPT)])


@functools.cache
def _make_sc_edge():
  return pl.kernel(
      _sc_body,
      out_type=jax.ShapeDtypeStruct((_NC * _NPAD, _D), jnp.bfloat16),
      mesh=plsc.VectorSubcoreMesh(core_axis_name="c", subcore_axis_name="s"),
      compiler_params=pltpu.CompilerParams(needs_layout_passes=False, use_tc_tiling_on_sc=False),
      scratch_types=[
          pltpu.VMEM((_PKW,), jnp.int32),
          pltpu.VMEM((_PKW,), jnp.int32),
          pltpu.VMEM((_CH, _D), jnp.float32),
          pltpu.VMEM((_CH, _D), jnp.float32),
          pltpu.VMEM((_CH, 2 * _D), jnp.float32),
          pltpu.VMEM((_CH, 2 * _D), jnp.float32),
          pltpu.VMEM((_CH,), jnp.int32),
          pltpu.VMEM((_CH, _D), jnp.bfloat16),
          pltpu.VMEM((272,), jnp.float32),
          pltpu.VMEM_SHARED((_NPAD, _D), jnp.bfloat16),
          pltpu.SemaphoreType.DMA,
          pltpu.SemaphoreType.DMA,
          pltpu.SemaphoreType.DMA,
          pltpu.SemaphoreType.DMA,
          pltpu.SemaphoreType.DMA,
          pltpu.SemaphoreType.DMA,
          pltpu.SemaphoreType.DMA,
          pltpu.SemaphoreType.DMA,
      ],
  )


def _epilogue_body(p0_ref, p1_ref, x_ref, pm_ref, g_ref, b_ref, o_ref):
  p = (p0_ref[...] + p1_ref[...]).astype(jnp.float32)
  o = jnp.dot(p, pm_ref[...], preferred_element_type=jnp.float32) * 0.01 \
      + x_ref[...]
  mean = jnp.mean(o, axis=1, keepdims=True)
  co = o - mean
  var = jnp.mean(co * co, axis=1, keepdims=True)
  ln = co * jax.lax.rsqrt(var + 1e-5) * g_ref[...] + b_ref[...]
  o_ref[...] = ln / (1.0 + jnp.exp(-ln))


def _epilogue(p0, p1, x, pm, gamma, beta):
  return pl.pallas_call(
      _epilogue_body,
      grid=(_NBLK,),
      in_specs=[
          pl.BlockSpec((_RBLK, _D), lambda i: (i, 0)),
          pl.BlockSpec((_RBLK, _D), lambda i: (i, 0)),
          pl.BlockSpec((_RBLK, _D), lambda i: (i, 0)),
          pl.BlockSpec((_D, _D), lambda i: (0, 0)),
          pl.BlockSpec((1, _D), lambda i: (0, 0)),
          pl.BlockSpec((1, _D), lambda i: (0, 0)),
      ],
      out_specs=pl.BlockSpec((_RBLK, _D), lambda i: (i, 0)),
      out_shape=jax.ShapeDtypeStruct((_N, _D), jnp.float32),
  )(p0, p1, x, pm, gamma, beta)


def kernel(h, distances, edges, node_mask, edge_mask, W_lin, b_lin, W1, b1,
           W2, b2, gamma, beta):
  x, a, bx = _prologue(h, W_lin, b_lin.reshape(1, _D), W1[:_D], W1[_D:2 * _D],
                       b1.reshape(1, _D))
  row = edges[0].astype(jnp.int32)
  col = edges[1].astype(jnp.int32)
  db = jnp.broadcast_to(distances, (_E, 16))
  dbi = jax.lax.bitcast_convert_type(db, jnp.int32)
  pk = jnp.concatenate(
      [row.reshape(-1, _CH), col.reshape(-1, _CH),
       dbi.reshape(-1, _CH * 16)], axis=1).reshape(-1)
  params = jnp.concatenate(
      [W1[2 * _D], W2[:, 0], jnp.pad(b2, (0, 15))]).astype(jnp.float32)
  zeros = jnp.zeros((_NPAD, _D), jnp.bfloat16)
  parts = _make_sc_edge()(pk, row, a, bx, params, zeros)
  blk = jnp.arange(128) // 32
  pos = jnp.arange(128) % 32
  feat = blk * 32 + jnp.where(pos % 2 == 0, pos // 2, 16 + pos // 2)
  pm = jax.nn.one_hot(feat, _D, dtype=jnp.float32)
  h_out = _epilogue(parts[:_N], parts[_NPAD:_NPAD + _N], x, pm,
                    gamma.reshape(1, _D), beta.reshape(1, _D))
  return (h_out, distances, edges, node_mask, edge_mask)


# speedup vs baseline: 1.4945x; 1.3965x over previous
import functools

import jax
import jax.numpy as jnp
from jax import lax
from jax.experimental import pallas as pl
from jax.experimental.pallas import tpu as pltpu
from jax.experimental.pallas import tpu_sc as plsc

_N = 10000
_E = 320000
_D = 128

_NC = 2
_NS = 16
_NW = _NC * _NS
_EPW = _E // _NW
_CH = 80
_NCH = _EPW // _CH
_NPAD = 10240
_RPT = _NPAD // _NS

_RBLK = 1000
_NBLK = _N // _RBLK


def _prologue_body(h_ref, wl_ref, bl_ref, w1a_ref, w1b_ref, b1_ref,
                   x_ref, a_ref, bx_ref):
  x = jnp.dot(h_ref[...], wl_ref[...],
              preferred_element_type=jnp.float32) + bl_ref[...]
  x_ref[...] = x
  a_ref[...] = jnp.dot(x, w1a_ref[...],
                       preferred_element_type=jnp.float32) + b1_ref[...]
  b = jnp.dot(x, w1b_ref[...], preferred_element_type=jnp.float32)
  bx_ref[...] = jnp.concatenate([b, x], axis=1)


def _prologue(h, wl, bl, w1a, w1b, b1):
  return pl.pallas_call(
      _prologue_body,
      grid=(_NBLK,),
      in_specs=[
          pl.BlockSpec((_RBLK, _D), lambda i: (i, 0)),
          pl.BlockSpec((_D, _D), lambda i: (0, 0)),
          pl.BlockSpec((1, _D), lambda i: (0, 0)),
          pl.BlockSpec((_D, _D), lambda i: (0, 0)),
          pl.BlockSpec((_D, _D), lambda i: (0, 0)),
          pl.BlockSpec((1, _D), lambda i: (0, 0)),
      ],
      out_specs=[
          pl.BlockSpec((_RBLK, _D), lambda i: (i, 0)),
          pl.BlockSpec((_RBLK, _D), lambda i: (i, 0)),
          pl.BlockSpec((_RBLK, 2 * _D), lambda i: (i, 0)),
      ],
      out_shape=[
          jax.ShapeDtypeStruct((_N, _D), jnp.float32),
          jax.ShapeDtypeStruct((_N, _D), jnp.float32),
          jax.ShapeDtypeStruct((_N, 2 * _D), jnp.float32),
      ],
  )(h, wl, bl, w1a, w1b, b1)


def _sc_body(row_h, col_h, dist_h, a_h, bx_h, par_h, zer_h, out_h,
             row_v0, row_v1, col_v0, col_v1, dist_v0, dist_v1,
             ar0, ar1, bxr0, bxr1, rsc, agg, par_v, acc_sh,
             sem_r0, sem_r1, sem_c0, sem_c1, sem_d0, sem_d1,
             sem_ga0, sem_ga1, sem_gb0, sem_gb1, sem_sc):
  row_v = [row_v0, row_v1]
  col_v = [col_v0, col_v1]
  dist_v = [dist_v0, dist_v1]
  arows = [ar0, ar1]
  bxrows = [bxr0, bxr1]
  sem_r = [sem_r0, sem_r1]
  sem_c = [sem_c0, sem_c1]
  sem_d = [sem_d0, sem_d1]
  sem_ga = [sem_ga0, sem_ga1]
  sem_gb = [sem_gb0, sem_gb1]

  c = lax.axis_index("c")
  s = lax.axis_index("s")
  wid = c * _NS + s

  pltpu.sync_copy(zer_h.at[pl.ds(s * _RPT, _RPT)],
                  acc_sh.at[pl.ds(s * _RPT, _RPT)])
  pltpu.sync_copy(par_h, par_v)
  plsc.subcore_barrier()

  wd = [par_v[pl.ds(16 * k, 16)] for k in range(8)]
  w2 = [par_v[pl.ds(128 + 16 * k, 16)] for k in range(8)]
  b2 = par_v[pl.ds(256, 16)][0]

  ebase = wid * _EPW

  def issue_idx(ci, p):
    base = ebase + ci * _CH
    pltpu.async_copy(row_h.at[pl.ds(base, _CH)], row_v[p], sem_r[p])
    pltpu.async_copy(col_h.at[pl.ds(base, _CH)], col_v[p], sem_c[p])
    pltpu.async_copy(dist_h.at[pl.ds(base, _CH)], dist_v[p], sem_d[p])

  def wait_idx(ci, p):
    base = ebase + ci * _CH
    pltpu.make_async_copy(row_h.at[pl.ds(base, _CH)], row_v[p],
                          sem_r[p]).wait()
    pltpu.make_async_copy(col_h.at[pl.ds(base, _CH)], col_v[p],
                          sem_c[p]).wait()
    pltpu.make_async_copy(dist_h.at[pl.ds(base, _CH)], dist_v[p],
                          sem_d[p]).wait()

  def issue_gather(p):
    pltpu.async_copy(a_h.at[row_v[p]], arows[p], sem_ga[p])
    pltpu.async_copy(bx_h.at[col_v[p]], bxrows[p], sem_gb[p])

  def wait_gather(p):
    pltpu.make_async_copy(a_h.at[row_v[p]], arows[p], sem_ga[p]).wait()
    pltpu.make_async_copy(bx_h.at[col_v[p]], bxrows[p], sem_gb[p]).wait()

  def compute(p):
    av = arows[p]
    bv = bxrows[p]
    dvr = dist_v[p]

    @plsc.parallel_loop(0, _CH, unroll=1)
    def _edge(e):
      g = (e // 16) * 16
      dvec = dvr[pl.ds(g, 16)]
      lane = jnp.full((16,), e - g, jnp.int32)
      dv = dvec.at[lane].get(mode="promise_in_bounds")
      acc0 = jnp.zeros((16,), jnp.float32)
      acc1 = jnp.zeros((16,), jnp.float32)
      for k in range(8):
        pre = av[e, pl.ds(16 * k, 16)] + bv[e, pl.ds(16 * k, 16)] + dv * wd[k]
        sl = pre / (1.0 + jnp.exp(-pre))
        if k % 2 == 0:
          acc0 = acc0 + sl * w2[k]
        else:
          acc1 = acc1 + sl * w2[k]
      t = jnp.sum(acc0 + acc1) + b2
      attv = 1.0 / (1.0 + jnp.exp(jnp.full((16,), 0.0, jnp.float32) - t))
      for k2 in range(4):
        xa = bv[e, pl.ds(128 + 32 * k2, 16)] * attv
        xb = bv[e, pl.ds(128 + 32 * k2 + 16, 16)] * attv
        agg[e, pl.ds(32 * k2, 32)] = plsc.pack(
            xa, xb, format=plsc.PackFormat.INTERLEAVED)

  def wait_sc():
    pltpu.make_async_copy(agg, acc_sh.at[rsc], sem_sc).wait()

  issue_idx(0, 0)
  wait_idx(0, 0)
  issue_gather(0)
  issue_idx(1, 1)

  @pl.loop(0, _NCH // 2)
  def _outer(co):
    for b in range(2):
      p = b
      q = 1 - b
      ci = 2 * co + b

      @pl.when(ci >= 1)
      def _():
        wait_sc()

      @pl.when(ci + 1 < _NCH)
      def _():
        wait_idx(ci + 1, q)
        issue_gather(q)

      wait_gather(p)
      compute(p)
      for i in range(_CH // 16):
        rsc[pl.ds(16 * i, 16)] = row_v[p][pl.ds(16 * i, 16)]
      pltpu.async_copy(agg, acc_sh.at[rsc], sem_sc, add=True)

      @pl.when(ci + 2 < _NCH)
      def _():
        issue_idx(ci + 2, p)

  wait_sc()
  plsc.subcore_barrier()
  pltpu.sync_copy(acc_sh.at[pl.ds(s * _RPT, _RPT)],
                  out_h.at[pl.ds(c * _NPAD + s * _RPT, _RPT)])


@functools.cache
def _make_sc_edge():
  return pl.kernel(
      _sc_body,
      out_type=jax.ShapeDtypeStruct((_NC * _NPAD, _D), jnp.bfloat16),
      mesh=plsc.VectorSubcoreMesh(core_axis_name="c", subcore_axis_name="s"),
      compiler_params=pltpu.CompilerParams(needs_layout_passes=False,
                                           use_tc_tiling_on_sc=False),
      scratch_types=(
          [pltpu.VMEM((_CH,), jnp.int32)] * 4
          + [pltpu.VMEM((_CH,), jnp.float32)] * 2
          + [
              pltpu.VMEM((_CH, _D), jnp.float32),
              pltpu.VMEM((_CH, _D), jnp.float32),
              pltpu.VMEM((_CH, 2 * _D), jnp.float32),
              pltpu.VMEM((_CH, 2 * _D), jnp.float32),
              pltpu.VMEM((_CH,), jnp.int32),
              pltpu.VMEM((_CH, _D), jnp.bfloat16),
              pltpu.VMEM((272,), jnp.float32),
              pltpu.VMEM_SHARED((_NPAD, _D), jnp.bfloat16),
          ]
          + [pltpu.SemaphoreType.DMA] * 11
      ),
  )


def _epilogue_body(p0_ref, p1_ref, x_ref, pm_ref, g_ref, b_ref, o_ref):
  p = (p0_ref[...] + p1_ref[...]).astype(jnp.float32)
  o = jnp.dot(p, pm_ref[...], preferred_element_type=jnp.float32) * 0.01 \
      + x_ref[...]
  mean = jnp.mean(o, axis=1, keepdims=True)
  co = o - mean
  var = jnp.mean(co * co, axis=1, keepdims=True)
  ln = co * jax.lax.rsqrt(var + 1e-5) * g_ref[...] + b_ref[...]
  o_ref[...] = ln / (1.0 + jnp.exp(-ln))


def _epilogue(p0, p1, x, pm, gamma, beta):
  return pl.pallas_call(
      _epilogue_body,
      grid=(_NBLK,),
      in_specs=[
          pl.BlockSpec((_RBLK, _D), lambda i: (i, 0)),
          pl.BlockSpec((_RBLK, _D), lambda i: (i, 0)),
          pl.BlockSpec((_RBLK, _D), lambda i: (i, 0)),
          pl.BlockSpec((_D, _D), lambda i: (0, 0)),
          pl.BlockSpec((1, _D), lambda i: (0, 0)),
          pl.BlockSpec((1, _D), lambda i: (0, 0)),
      ],
      out_specs=pl.BlockSpec((_RBLK, _D), lambda i: (i, 0)),
      out_shape=jax.ShapeDtypeStruct((_N, _D), jnp.float32),
  )(p0, p1, x, pm, gamma, beta)


def kernel(h, distances, edges, node_mask, edge_mask, W_lin, b_lin, W1, b1,
           W2, b2, gamma, beta):
  x, a, bx = _prologue(h, W_lin, b_lin.reshape(1, _D), W1[:_D], W1[_D:2 * _D],
                       b1.reshape(1, _D))
  row = edges[0].astype(jnp.int32)
  col = edges[1].astype(jnp.int32)
  dist = distances.reshape(-1)
  params = jnp.concatenate(
      [W1[2 * _D], W2[:, 0], jnp.pad(b2, (0, 15))]).astype(jnp.float32)
  zeros = jnp.zeros((_NPAD, _D), jnp.bfloat16)
  parts = _make_sc_edge()(row, col, dist, a, bx, params, zeros)
  blk = jnp.arange(128) // 32
  pos = jnp.arange(128) % 32
  feat = blk * 32 + jnp.where(pos % 2 == 0, pos // 2, 16 + pos // 2)
  pm = jax.nn.one_hot(feat, _D, dtype=jnp.float32)
  h_out = _epilogue(parts[:_N], parts[_NPAD:_NPAD + _N], x, pm,
                    gamma.reshape(1, _D), beta.reshape(1, _D))
  return (h_out, distances, edges, node_mask, edge_mask)
